# baseline (speedup 1.0000x reference)
import functools

import jax
import jax.numpy as jnp
from jax import lax
from jax.experimental import pallas as pl
from jax.experimental.pallas import tpu as pltpu
from jax.experimental.pallas import tpu_sc as plsc

NUM_USERS = 60000
NUM_ITEMS = 39000
N_TOTAL = 100000
EMBED_DIM = 32
NUM_LAYERS = 2
BATCH = 4096
N_EDGES = 1600000

NC = 2
NS = 16
HALF = 16
LANES = 16

IW = 128
E_PAD = 1605632
E_ROWS = E_PAD // IW
ROWS_PER_TEC = E_ROWS // NS
CROWS = 4
CHUNK_E = CROWS * IW
NCHUNKS = ROWS_PER_TEC // CROWS
N_PAD = 100096
STRIPE = N_PAD // NS
BPT = BATCH // NS
BROWS = BPT // IW

_mesh = plsc.VectorSubcoreMesh(core_axis_name="c", subcore_axis_name="s")


@functools.partial(
    pl.kernel,
    out_type=(
        jax.ShapeDtypeStruct((NC * BATCH, HALF), jnp.float32),
        jax.ShapeDtypeStruct((NC * BATCH, HALF), jnp.float32),
        jax.ShapeDtypeStruct((NC * N_PAD, HALF), jnp.float32),
    ),
    mesh=_mesh,
    compiler_params=pltpu.CompilerParams(use_tc_tiling_on_sc=False),
    scratch_types=[
        pltpu.VMEM_SHARED((N_PAD, HALF), jnp.float32),
        pltpu.VMEM((CROWS, IW), jnp.int32),
        pltpu.VMEM((CROWS, IW), jnp.int32),
        pltpu.VMEM((CROWS, IW), jnp.float32),
        pltpu.VMEM((CROWS, IW), jnp.int32),
        pltpu.VMEM((CROWS, IW), jnp.int32),
        pltpu.VMEM((CROWS, IW), jnp.float32),
        pltpu.VMEM((CROWS, IW), jnp.int32),
        pltpu.VMEM((CROWS, IW), jnp.int32),
        pltpu.VMEM((CHUNK_E, HALF), jnp.float32),
        pltpu.VMEM((CHUNK_E, HALF), jnp.float32),
        pltpu.VMEM((BROWS, IW), jnp.int32),
        pltpu.VMEM((BROWS, IW), jnp.int32),
        pltpu.VMEM((IW, HALF), jnp.float32),
        pltpu.VMEM((IW, HALF), jnp.float32),
        pltpu.VMEM((IW, HALF), jnp.float32),
        pltpu.SemaphoreType.DMA,
        pltpu.SemaphoreType.DMA,
        pltpu.SemaphoreType.DMA,
    ],
)
def _sc_propagate(emb_s, rows2, cols2, vals2, u2, i2,
                  ubuf, ibuf, e1s,
                  acc, colv0, rowv0, valv0, colv1, rowv1, valv1,
                  srow0, srow1, rowsv0, rowsv1, idxv, iadj,
                  tmp, fu, fi, sem, semg, sems):
    cid = lax.axis_index("c")
    sid = lax.axis_index("s")
    off = cid * N_PAD

    bufs = ((colv0, rowv0, valv0), (colv1, rowv1, valv1))
    srows = (srow0, srow1)
    rowsvs = (rowsv0, rowsv1)

    def fill_zero_rowsv():
        zero = jnp.zeros((LANES,), jnp.float32)
        def z(e, _):
            rowsv0[e, :] = zero
            return 0
        lax.fori_loop(0, CHUNK_E, z, 0)

    def zero_stripe():
        base = sid * STRIPE
        n_full = STRIPE // CHUNK_E
        rem = STRIPE - n_full * CHUNK_E
        for k in range(n_full):
            pltpu.sync_copy(rowsv0, acc.at[pl.ds(base + k * CHUNK_E, CHUNK_E)])
        pltpu.sync_copy(rowsv0.at[pl.ds(0, rem)],
                        acc.at[pl.ds(base + n_full * CHUNK_E, rem)])

    def add_offset(dst, src, n_rows, value):
        def oadd(t, _):
            j = t // 8
            l = pl.multiple_of((t % 8) * LANES, LANES)
            dst[j, pl.ds(l, LANES)] = src[j, pl.ds(l, LANES)] + value
            return 0
        lax.fori_loop(0, n_rows * 8, oadd, 0)

    def copy_rows(dst, src_):
        def cp(t, _):
            j = t // 8
            l = pl.multiple_of((t % 8) * LANES, LANES)
            dst[j, pl.ds(l, LANES)] = src_[j, pl.ds(l, LANES)]
            return 0
        lax.fori_loop(0, CROWS * 8, cp, 0)

    def fire_loads(ci, b):
        rbase = jnp.minimum(sid * ROWS_PER_TEC + ci * CROWS, E_ROWS - CROWS)
        col, row, val = bufs[b]
        pltpu.async_copy(cols2.at[pl.ds(rbase, CROWS)], col, sem)
        pltpu.async_copy(rows2.at[pl.ds(rbase, CROWS)], row, sem)
        pltpu.async_copy(vals2.at[pl.ds(rbase, CROWS)], val, sem)

    def drain_loads(b):
        col, row, val = bufs[b]
        pltpu.make_async_copy(cols2.at[pl.ds(0, CROWS)], col, sem).wait()
        pltpu.make_async_copy(rows2.at[pl.ds(0, CROWS)], row, sem).wait()
        pltpu.make_async_copy(vals2.at[pl.ds(0, CROWS)], val, sem).wait()

    def drain_scatters(b):
        R, S = rowsvs[b], srows[b]
        for j in range(CROWS):
            pltpu.make_async_copy(R.at[pl.ds(j * IW, IW)],
                                  acc.at[S.at[j]], sems).wait()

    def edge_pass(src):
        def chunk_body(ci, b, drain_scat):
            colc, rowc, valc = bufs[b]
            coln = bufs[1 - b][0]
            R, S = rowsvs[b], srows[b]
            if drain_scat:
                drain_scatters(b)
            gd = [pltpu.async_copy(src.at[colc.at[j]],
                                   R.at[pl.ds(j * IW, IW)], semg)
                  for j in range(CROWS)]
            drain_loads(1 - b)
            add_offset(coln, coln, CROWS, off)
            copy_rows(S, rowc)
            for j in range(CROWS):
                gd[j].wait()
                def scale(g, _, j=j):
                    l = pl.multiple_of(g * LANES, LANES)
                    vv = valc[j, pl.ds(l, LANES)]
                    base_e = j * IW + g * LANES
                    for lane in range(LANES):
                        e = base_e + lane
                        R[e, :] = R[e, :] * vv[lane]
                    return 0
                lax.fori_loop(0, IW // LANES, scale, 0)
                pltpu.async_copy(R.at[pl.ds(j * IW, IW)],
                                 acc.at[S.at[j]], sems, add=True)
            fire_loads(ci + 2, b)

        fire_loads(0, 0)
        drain_loads(0)
        add_offset(colv0, colv0, CROWS, off)
        fire_loads(1, 1)
        chunk_body(0, 0, drain_scat=False)
        chunk_body(1, 1, drain_scat=False)

        def chunk_pair(p, _):
            chunk_body(2 * p, 0, drain_scat=True)
            chunk_body(2 * p + 1, 1, drain_scat=True)
            return 0
        lax.fori_loop(1, NCHUNKS // 2, chunk_pair, 0)
        drain_scatters(0)
        drain_scatters(1)
        drain_loads(1)

    def dump_stripe(dst):
        base = sid * STRIPE
        pltpu.sync_copy(acc.at[pl.ds(base, STRIPE)],
                        dst.at[pl.ds(off + base, STRIPE)])

    def batch_out():
        pltpu.sync_copy(u2.at[pl.ds(sid * BROWS, BROWS)], idxv)
        add_offset(iadj, idxv, BROWS, off)
        for j in range(BROWS):
            _final_block(j, fu, ubuf)
        pltpu.sync_copy(i2.at[pl.ds(sid * BROWS, BROWS)], idxv)
        add_offset(iadj, idxv, BROWS, off)
        for j in range(BROWS):
            _final_block(j, fi, ibuf)

    def _final_block(j, fbuf, obuf):
        def accum(e, _):
            fbuf[e, :] = fbuf[e, :] + tmp[e, :]
            return 0
        pltpu.sync_copy(emb_s.at[iadj.at[j]], fbuf)
        pltpu.sync_copy(e1s.at[iadj.at[j]], tmp)
        lax.fori_loop(0, IW, accum, 0)
        pltpu.sync_copy(acc.at[idxv.at[j]], tmp)
        lax.fori_loop(0, IW, accum, 0)
        obase = cid * BATCH + sid * BPT + j * IW
        pltpu.sync_copy(fbuf, obuf.at[pl.ds(obase, IW)])

    fill_zero_rowsv()
    zero_stripe()
    plsc.subcore_barrier()
    edge_pass(emb_s)
    plsc.subcore_barrier()
    dump_stripe(e1s)
    fill_zero_rowsv()
    zero_stripe()
    plsc.subcore_barrier()
    edge_pass(e1s)
    plsc.subcore_barrier()
    batch_out()


def kernel(user_ids, item_ids, node_emb, adj_row, adj_col, adj_vals):
    npad = N_PAD - N_TOTAL
    emb_s = jnp.concatenate(
        [jnp.pad(node_emb[:, :HALF], ((0, npad), (0, 0))),
         jnp.pad(node_emb[:, HALF:], ((0, npad), (0, 0)))], axis=0)
    pad = E_PAD - N_EDGES
    rows2 = jnp.pad(adj_row.astype(jnp.int32), (0, pad)).reshape(E_ROWS, IW)
    cols2 = jnp.pad(adj_col.astype(jnp.int32), (0, pad)).reshape(E_ROWS, IW)
    vals2 = jnp.pad(adj_vals, (0, pad)).reshape(E_ROWS, IW)
    u2 = user_ids.astype(jnp.int32).reshape(BATCH // IW, IW)
    i2 = (item_ids.astype(jnp.int32) + NUM_USERS).reshape(BATCH // IW, IW)
    ubuf, ibuf, _e1 = _sc_propagate(emb_s, rows2, cols2, vals2, u2, i2)
    part = pl.pallas_call(
        _dot_body,
        out_shape=jax.ShapeDtypeStruct((NC * BATCH,), jnp.float32),
    )(ubuf, ibuf)
    return part[:BATCH] + part[BATCH:]


def _dot_body(u_ref, i_ref, o_ref):
    o_ref[...] = jnp.sum(u_ref[...] * i_ref[...], axis=1) * (1.0 / 9.0)


# speedup vs baseline: 1.1824x; 1.1824x over previous
import functools

import jax
import jax.numpy as jnp
from jax import lax
from jax.experimental import pallas as pl
from jax.experimental.pallas import tpu as pltpu
from jax.experimental.pallas import tpu_sc as plsc

NUM_USERS = 60000
NUM_ITEMS = 39000
N_TOTAL = 100000
EMBED_DIM = 32
NUM_LAYERS = 2
BATCH = 4096
N_EDGES = 1600000

NC = 2
NS = 16
HALF = 16
LANES = 16

IW = 128
E_PAD = 1605632
E_ROWS = E_PAD // IW
ROWS_PER_TEC = E_ROWS // NS
CROWS = 8
CHUNK_E = CROWS * IW
NCHUNKS = ROWS_PER_TEC // CROWS
N_PAD = 100096
STRIPE = N_PAD // NS
BPT = BATCH // NS
BROWS = BPT // IW

_mesh = plsc.VectorSubcoreMesh(core_axis_name="c", subcore_axis_name="s")


@functools.partial(
    pl.kernel,
    out_type=(
        jax.ShapeDtypeStruct((NC * BATCH, HALF), jnp.float32),
        jax.ShapeDtypeStruct((NC * BATCH, HALF), jnp.float32),
        jax.ShapeDtypeStruct((NC * N_PAD, HALF), jnp.float32),
    ),
    mesh=_mesh,
    compiler_params=pltpu.CompilerParams(use_tc_tiling_on_sc=False),
    scratch_types=[
        pltpu.VMEM_SHARED((N_PAD, HALF), jnp.float32),
        pltpu.VMEM((CROWS, IW), jnp.int32),
        pltpu.VMEM((CROWS, IW), jnp.int32),
        pltpu.VMEM((CROWS, IW), jnp.float32),
        pltpu.VMEM((CROWS, IW), jnp.int32),
        pltpu.VMEM((CROWS, IW), jnp.int32),
        pltpu.VMEM((CROWS, IW), jnp.float32),
        pltpu.VMEM((CHUNK_E, HALF), jnp.float32),
        pltpu.VMEM((BROWS, IW), jnp.int32),
        pltpu.VMEM((BROWS, IW), jnp.int32),
        pltpu.VMEM((IW, HALF), jnp.float32),
        pltpu.VMEM((IW, HALF), jnp.float32),
        pltpu.VMEM((IW, HALF), jnp.float32),
        pltpu.SemaphoreType.DMA,
        pltpu.SemaphoreType.DMA,
        pltpu.SemaphoreType.DMA,
    ],
)
def _sc_propagate(emb_s, rows2, cols2, vals2, u2, i2,
                  ubuf, ibuf, e1s,
                  acc, colv0, rowv0, valv0, colv1, rowv1, valv1,
                  rowsv0, idxv, iadj,
                  tmp, fu, fi, sem, semg, sems):
    cid = lax.axis_index("c")
    sid = lax.axis_index("s")
    off = cid * N_PAD

    bufs = ((colv0, rowv0, valv0), (colv1, rowv1, valv1))

    def fill_zero_rowsv():
        zero = jnp.zeros((LANES,), jnp.float32)
        def z(e, _):
            rowsv0[e, :] = zero
            return 0
        lax.fori_loop(0, CHUNK_E, z, 0)

    def zero_stripe():
        base = sid * STRIPE
        n_full = STRIPE // CHUNK_E
        rem = STRIPE - n_full * CHUNK_E
        for k in range(n_full):
            pltpu.sync_copy(rowsv0, acc.at[pl.ds(base + k * CHUNK_E, CHUNK_E)])
        pltpu.sync_copy(rowsv0.at[pl.ds(0, rem)],
                        acc.at[pl.ds(base + n_full * CHUNK_E, rem)])

    def add_offset(dst, src, n_rows, value):
        def oadd(t, _):
            j = t // 8
            l = pl.multiple_of((t % 8) * LANES, LANES)
            dst[j, pl.ds(l, LANES)] = src[j, pl.ds(l, LANES)] + value
            return 0
        lax.fori_loop(0, n_rows * 8, oadd, 0)

    def fire_loads(ci, b):
        rbase = jnp.minimum(sid * ROWS_PER_TEC + ci * CROWS, E_ROWS - CROWS)
        col, row, val = bufs[b]
        pltpu.async_copy(cols2.at[pl.ds(rbase, CROWS)], col, sem)
        pltpu.async_copy(rows2.at[pl.ds(rbase, CROWS)], row, sem)
        pltpu.async_copy(vals2.at[pl.ds(rbase, CROWS)], val, sem)

    def drain_loads(b):
        col, row, val = bufs[b]
        pltpu.make_async_copy(cols2.at[pl.ds(0, CROWS)], col, sem).wait()
        pltpu.make_async_copy(rows2.at[pl.ds(0, CROWS)], row, sem).wait()
        pltpu.make_async_copy(vals2.at[pl.ds(0, CROWS)], val, sem).wait()

    def edge_pass(src):
        def half_body(ci, b):
            colc, rowc, valc = bufs[b]
            coln = bufs[1 - b][0]
            gd = [pltpu.async_copy(src.at[colc.at[j]],
                                   rowsv0.at[pl.ds(j * IW, IW)], semg)
                  for j in range(CROWS)]
            drain_loads(1 - b)
            add_offset(coln, coln, CROWS, off)
            sd = []
            for j in range(CROWS):
                gd[j].wait()
                @plsc.parallel_loop(0, IW // LANES, unroll=4)
                def scale(g, j=j):
                    l = pl.multiple_of(g * LANES, LANES)
                    vv = valc[j, pl.ds(l, LANES)]
                    base_e = j * IW + g * LANES
                    for lane in range(LANES):
                        e = base_e + lane
                        rowsv0[e, :] = rowsv0[e, :] * vv[lane]
                sd.append(pltpu.async_copy(rowsv0.at[pl.ds(j * IW, IW)],
                                           acc.at[rowc.at[j]], sems, add=True))
            for d in sd:
                d.wait()
            fire_loads(ci + 2, b)

        fire_loads(0, 0)
        drain_loads(0)
        add_offset(colv0, colv0, CROWS, off)
        fire_loads(1, 1)

        def chunk_pair(p, _):
            half_body(2 * p, 0)
            half_body(2 * p + 1, 1)
            return 0
        lax.fori_loop(0, NCHUNKS // 2, chunk_pair, 0)
        drain_loads(1)

    def dump_stripe(dst):
        base = sid * STRIPE
        pltpu.sync_copy(acc.at[pl.ds(base, STRIPE)],
                        dst.at[pl.ds(off + base, STRIPE)])

    def batch_out():
        pltpu.sync_copy(u2.at[pl.ds(sid * BROWS, BROWS)], idxv)
        add_offset(iadj, idxv, BROWS, off)
        for j in range(BROWS):
            _final_block(j, fu, ubuf)
        pltpu.sync_copy(i2.at[pl.ds(sid * BROWS, BROWS)], idxv)
        add_offset(iadj, idxv, BROWS, off)
        for j in range(BROWS):
            _final_block(j, fi, ibuf)

    def _final_block(j, fbuf, obuf):
        def accum(e, _):
            fbuf[e, :] = fbuf[e, :] + tmp[e, :]
            return 0
        pltpu.sync_copy(emb_s.at[iadj.at[j]], fbuf)
        pltpu.sync_copy(e1s.at[iadj.at[j]], tmp)
        lax.fori_loop(0, IW, accum, 0)
        pltpu.sync_copy(acc.at[idxv.at[j]], tmp)
        lax.fori_loop(0, IW, accum, 0)
        obase = cid * BATCH + sid * BPT + j * IW
        pltpu.sync_copy(fbuf, obuf.at[pl.ds(obase, IW)])

    fill_zero_rowsv()
    zero_stripe()
    plsc.subcore_barrier()
    edge_pass(emb_s)
    plsc.subcore_barrier()
    dump_stripe(e1s)
    fill_zero_rowsv()
    zero_stripe()
    plsc.subcore_barrier()
    edge_pass(e1s)
    plsc.subcore_barrier()
    batch_out()


def kernel(user_ids, item_ids, node_emb, adj_row, adj_col, adj_vals):
    npad = N_PAD - N_TOTAL
    emb_s = jnp.concatenate(
        [jnp.pad(node_emb[:, :HALF], ((0, npad), (0, 0))),
         jnp.pad(node_emb[:, HALF:], ((0, npad), (0, 0)))], axis=0)
    pad = E_PAD - N_EDGES
    rows2 = jnp.pad(adj_row.astype(jnp.int32), (0, pad)).reshape(E_ROWS, IW)
    cols2 = jnp.pad(adj_col.astype(jnp.int32), (0, pad)).reshape(E_ROWS, IW)
    vals2 = jnp.pad(adj_vals, (0, pad)).reshape(E_ROWS, IW)
    u2 = user_ids.astype(jnp.int32).reshape(BATCH // IW, IW)
    i2 = (item_ids.astype(jnp.int32) + NUM_USERS).reshape(BATCH // IW, IW)
    ubuf, ibuf, _e1 = _sc_propagate(emb_s, rows2, cols2, vals2, u2, i2)
    part = pl.pallas_call(
        _dot_body,
        out_shape=jax.ShapeDtypeStruct((NC * BATCH,), jnp.float32),
    )(ubuf, ibuf)
    return part[:BATCH] + part[BATCH:]


def _dot_body(u_ref, i_ref, o_ref):
    o_ref[...] = jnp.sum(u_ref[...] * i_ref[...], axis=1) * (1.0 / 9.0)


# speedup vs baseline: 1.2061x; 1.0200x over previous
import functools

import jax
import jax.numpy as jnp
from jax import lax
from jax.experimental import pallas as pl
from jax.experimental.pallas import tpu as pltpu
from jax.experimental.pallas import tpu_sc as plsc

NUM_USERS = 60000
NUM_ITEMS = 39000
N_TOTAL = 100000
EMBED_DIM = 32
NUM_LAYERS = 2
BATCH = 4096
N_EDGES = 1600000

NC = 2
NS = 16
HALF = 16
LANES = 16

IW = 128
E_PAD = 1605632
E_ROWS = E_PAD // IW
ROWS_PER_TEC = E_ROWS // NS
CROWS = 8
CHUNK_E = CROWS * IW
NCHUNKS = ROWS_PER_TEC // CROWS
N_PAD = 100096
STRIPE = N_PAD // NS
BPT = BATCH // NS
BROWS = BPT // IW

_mesh = plsc.VectorSubcoreMesh(core_axis_name="c", subcore_axis_name="s")


@functools.partial(
    pl.kernel,
    out_type=(
        jax.ShapeDtypeStruct((NC * BATCH, HALF), jnp.float32),
        jax.ShapeDtypeStruct((NC * BATCH, HALF), jnp.float32),
        jax.ShapeDtypeStruct((NC * N_PAD, HALF), jnp.float32),
    ),
    mesh=_mesh,
    compiler_params=pltpu.CompilerParams(use_tc_tiling_on_sc=False),
    scratch_types=[
        pltpu.VMEM_SHARED((N_PAD, HALF), jnp.float32),
        pltpu.VMEM((CROWS, IW), jnp.int32),
        pltpu.VMEM((CROWS, IW), jnp.int32),
        pltpu.VMEM((CROWS, IW), jnp.float32),
        pltpu.VMEM((CROWS, IW), jnp.int32),
        pltpu.VMEM((CROWS, IW), jnp.int32),
        pltpu.VMEM((CROWS, IW), jnp.float32),
        pltpu.VMEM((CROWS, IW), jnp.int32),
        pltpu.VMEM((CROWS, IW), jnp.int32),
        pltpu.VMEM((CHUNK_E, HALF), jnp.float32),
        pltpu.VMEM((BROWS, IW), jnp.int32),
        pltpu.VMEM((BROWS, IW), jnp.int32),
        pltpu.VMEM((IW, HALF), jnp.float32),
        pltpu.VMEM((IW, HALF), jnp.float32),
        pltpu.SemaphoreType.DMA,
        pltpu.SemaphoreType.DMA,
        pltpu.SemaphoreType.DMA,
    ],
)
def _sc_propagate(emb_s, rows2, cols2, vals2, u2, i2,
                  ubuf, ibuf, e1s,
                  acc, colv0, rowv0, valv0, colv1, rowv1, valv1,
                  srow0, srow1, rowsv0, idxv, iadj,
                  tmp, fbuf, sem, semg, sems):
    cid = lax.axis_index("c")
    sid = lax.axis_index("s")
    off = cid * N_PAD

    bufs = ((colv0, rowv0, valv0), (colv1, rowv1, valv1))
    srows = (srow0, srow1)

    def fill_zero_rowsv():
        zero = jnp.zeros((LANES,), jnp.float32)
        def z(e, _):
            rowsv0[e, :] = zero
            return 0
        lax.fori_loop(0, CHUNK_E, z, 0)

    def zero_stripe():
        base = sid * STRIPE
        n_full = STRIPE // CHUNK_E
        rem = STRIPE - n_full * CHUNK_E
        for k in range(n_full):
            pltpu.sync_copy(rowsv0, acc.at[pl.ds(base + k * CHUNK_E, CHUNK_E)])
        pltpu.sync_copy(rowsv0.at[pl.ds(0, rem)],
                        acc.at[pl.ds(base + n_full * CHUNK_E, rem)])

    def add_offset(dst, src, n_rows, value):
        def oadd(t, _):
            j = t // 8
            l = pl.multiple_of((t % 8) * LANES, LANES)
            dst[j, pl.ds(l, LANES)] = src[j, pl.ds(l, LANES)] + value
            return 0
        lax.fori_loop(0, n_rows * 8, oadd, 0)

    def fire_loads(ci, b):
        rbase = jnp.minimum(sid * ROWS_PER_TEC + ci * CROWS, E_ROWS - CROWS)
        col, row, val = bufs[b]
        pltpu.async_copy(cols2.at[pl.ds(rbase, CROWS)], col, sem)
        pltpu.async_copy(rows2.at[pl.ds(rbase, CROWS)], row, sem)
        pltpu.async_copy(vals2.at[pl.ds(rbase, CROWS)], val, sem)

    def drain_loads(b):
        col, row, val = bufs[b]
        pltpu.make_async_copy(cols2.at[pl.ds(0, CROWS)], col, sem).wait()
        pltpu.make_async_copy(rows2.at[pl.ds(0, CROWS)], row, sem).wait()
        pltpu.make_async_copy(vals2.at[pl.ds(0, CROWS)], val, sem).wait()

    def copy_rows(dst, src_):
        @plsc.parallel_loop(0, CROWS * 8, unroll=4)
        def cp(t):
            j = t // 8
            l = pl.multiple_of((t % 8) * LANES, LANES)
            dst[j, pl.ds(l, LANES)] = src_[j, pl.ds(l, LANES)]

    def edge_pass(src):
        def chunk_body(ci, b, first):
            colc, rowc, valc = bufs[b]
            coln = bufs[1 - b][0]
            S, Sp = srows[b], srows[1 - b]
            gd = []
            for j in range(CROWS):
                if not first:
                    pltpu.make_async_copy(rowsv0.at[pl.ds(j * IW, IW)],
                                          acc.at[Sp.at[j]], sems).wait()
                gd.append(pltpu.async_copy(src.at[colc.at[j]],
                                           rowsv0.at[pl.ds(j * IW, IW)], semg))
            drain_loads(1 - b)
            add_offset(coln, coln, CROWS, off)
            copy_rows(S, rowc)
            for j in range(CROWS):
                gd[j].wait()
                @plsc.parallel_loop(0, IW // LANES, unroll=4)
                def scale(g, j=j):
                    l = pl.multiple_of(g * LANES, LANES)
                    vv = valc[j, pl.ds(l, LANES)]
                    base_e = j * IW + g * LANES
                    for lane in range(LANES):
                        e = base_e + lane
                        rowsv0[e, :] = rowsv0[e, :] * vv[lane]
                pltpu.async_copy(rowsv0.at[pl.ds(j * IW, IW)],
                                 acc.at[S.at[j]], sems, add=True)
            fire_loads(ci + 2, b)

        fire_loads(0, 0)
        drain_loads(0)
        add_offset(colv0, colv0, CROWS, off)
        fire_loads(1, 1)
        chunk_body(0, 0, first=True)

        def chunk_pair(p, _):
            chunk_body(2 * p + 1, 1, first=False)
            chunk_body(2 * p + 2, 0, first=False)
            return 0
        lax.fori_loop(0, (NCHUNKS - 2) // 2, chunk_pair, 0)
        chunk_body(NCHUNKS - 1, 1, first=False)
        for j in range(CROWS):
            pltpu.make_async_copy(rowsv0.at[pl.ds(j * IW, IW)],
                                  acc.at[srow1.at[j]], sems).wait()
        drain_loads(1)

    def dump_stripe(dst):
        base = sid * STRIPE
        pltpu.sync_copy(acc.at[pl.ds(base, STRIPE)],
                        dst.at[pl.ds(off + base, STRIPE)])

    def batch_out():
        pltpu.sync_copy(u2.at[pl.ds(sid * BROWS, BROWS)], idxv)
        add_offset(iadj, idxv, BROWS, off)
        for j in range(BROWS):
            _final_block(j, ubuf)
        pltpu.sync_copy(i2.at[pl.ds(sid * BROWS, BROWS)], idxv)
        add_offset(iadj, idxv, BROWS, off)
        for j in range(BROWS):
            _final_block(j, ibuf)

    def _final_block(j, obuf):
        def accum(e, _):
            fbuf[e, :] = fbuf[e, :] + tmp[e, :]
            return 0
        pltpu.sync_copy(emb_s.at[iadj.at[j]], fbuf)
        pltpu.sync_copy(e1s.at[iadj.at[j]], tmp)
        lax.fori_loop(0, IW, accum, 0)
        pltpu.sync_copy(acc.at[idxv.at[j]], tmp)
        lax.fori_loop(0, IW, accum, 0)
        obase = cid * BATCH + sid * BPT + j * IW
        pltpu.sync_copy(fbuf, obuf.at[pl.ds(obase, IW)])

    fill_zero_rowsv()
    zero_stripe()
    plsc.subcore_barrier()
    edge_pass(emb_s)
    plsc.subcore_barrier()
    dump_stripe(e1s)
    fill_zero_rowsv()
    zero_stripe()
    plsc.subcore_barrier()
    edge_pass(e1s)
    plsc.subcore_barrier()
    batch_out()


def kernel(user_ids, item_ids, node_emb, adj_row, adj_col, adj_vals):
    npad = N_PAD - N_TOTAL
    emb_s = jnp.concatenate(
        [jnp.pad(node_emb[:, :HALF], ((0, npad), (0, 0))),
         jnp.pad(node_emb[:, HALF:], ((0, npad), (0, 0)))], axis=0)
    pad = E_PAD - N_EDGES
    rows2 = jnp.pad(adj_row.astype(jnp.int32), (0, pad)).reshape(E_ROWS, IW)
    cols2 = jnp.pad(adj_col.astype(jnp.int32), (0, pad)).reshape(E_ROWS, IW)
    vals2 = jnp.pad(adj_vals, (0, pad)).reshape(E_ROWS, IW)
    u2 = user_ids.astype(jnp.int32).reshape(BATCH // IW, IW)
    i2 = (item_ids.astype(jnp.int32) + NUM_USERS).reshape(BATCH // IW, IW)
    ubuf, ibuf, _e1 = _sc_propagate(emb_s, rows2, cols2, vals2, u2, i2)
    part = pl.pallas_call(
        _dot_body,
        out_shape=jax.ShapeDtypeStruct((NC * BATCH,), jnp.float32),
    )(ubuf, ibuf)
    return part[:BATCH] + part[BATCH:]


def _dot_body(u_ref, i_ref, o_ref):
    o_ref[...] = jnp.sum(u_ref[...] * i_ref[...], axis=1) * (1.0 / 9.0)


# speedup vs baseline: 1.2330x; 1.0224x over previous
import functools

import jax
import jax.numpy as jnp
from jax import lax
from jax.experimental import pallas as pl
from jax.experimental.pallas import tpu as pltpu
from jax.experimental.pallas import tpu_sc as plsc

NUM_USERS = 60000
NUM_ITEMS = 39000
N_TOTAL = 100000
EMBED_DIM = 32
NUM_LAYERS = 2
BATCH = 4096
N_EDGES = 1600000

NC = 2
NS = 16
HALF = 16
LANES = 16

IW = 128
E_PAD = 1605632
E_ROWS = E_PAD // IW
ROWS_PER_TEC = E_ROWS // NS
CROWS = 8
CHUNK_E = CROWS * IW
NCHUNKS = ROWS_PER_TEC // CROWS
N_PAD = 100096
STRIPE = N_PAD // NS
BPT = BATCH // NS
BROWS = BPT // IW

_mesh = plsc.VectorSubcoreMesh(core_axis_name="c", subcore_axis_name="s")


@functools.partial(
    pl.kernel,
    out_type=(
        jax.ShapeDtypeStruct((NC * BATCH, HALF), jnp.float32),
        jax.ShapeDtypeStruct((NC * BATCH, HALF), jnp.float32),
        jax.ShapeDtypeStruct((NC * N_PAD, HALF), jnp.float32),
    ),
    mesh=_mesh,
    compiler_params=pltpu.CompilerParams(use_tc_tiling_on_sc=False),
    scratch_types=[
        pltpu.VMEM_SHARED((N_PAD, HALF), jnp.float32),
        pltpu.VMEM((CROWS, IW), jnp.int32),
        pltpu.VMEM((CROWS, IW), jnp.int32),
        pltpu.VMEM((CROWS, IW), jnp.float32),
        pltpu.VMEM((CROWS, IW), jnp.int32),
        pltpu.VMEM((CROWS, IW), jnp.int32),
        pltpu.VMEM((CROWS, IW), jnp.float32),
        pltpu.VMEM((CROWS, IW), jnp.int32),
        pltpu.VMEM((CROWS, IW), jnp.int32),
        pltpu.VMEM((CHUNK_E, HALF), jnp.float32),
        pltpu.VMEM((BROWS, IW), jnp.int32),
        pltpu.VMEM((BROWS, IW), jnp.int32),
        pltpu.VMEM((IW, HALF), jnp.float32),
        pltpu.VMEM((IW, HALF), jnp.float32),
        pltpu.SemaphoreType.DMA,
        pltpu.SemaphoreType.DMA,
        pltpu.SemaphoreType.DMA,
    ],
)
def _sc_propagate(emb_s, rows2, cols2, vals2, u2, i2,
                  ubuf, ibuf, e1s,
                  acc, colv0, rowv0, valv0, colv1, rowv1, valv1,
                  srow0, srow1, rowsv0, idxv, iadj,
                  tmp, fbuf, sem, semg, sems):
    cid = lax.axis_index("c")
    sid = lax.axis_index("s")
    off = cid * N_PAD

    bufs = ((colv0, rowv0, valv0), (colv1, rowv1, valv1))
    srows = (srow0, srow1)

    def fill_zero_rowsv():
        zero = jnp.zeros((LANES,), jnp.float32)
        @plsc.parallel_loop(0, CHUNK_E, unroll=4)
        def z(e):
            rowsv0[e, :] = zero

    def zero_stripe():
        base = sid * STRIPE
        n_full = STRIPE // CHUNK_E
        rem = STRIPE - n_full * CHUNK_E
        ds = [pltpu.async_copy(rowsv0,
                               acc.at[pl.ds(base + k * CHUNK_E, CHUNK_E)], sem)
              for k in range(n_full)]
        ds.append(pltpu.async_copy(rowsv0.at[pl.ds(0, rem)],
                                   acc.at[pl.ds(base + n_full * CHUNK_E, rem)],
                                   sem))
        for d in ds:
            d.wait()

    def add_offset(dst, src, n_rows, value):
        def oadd(t, _):
            j = t // 8
            l = pl.multiple_of((t % 8) * LANES, LANES)
            dst[j, pl.ds(l, LANES)] = src[j, pl.ds(l, LANES)] + value
            return 0
        lax.fori_loop(0, n_rows * 8, oadd, 0)

    def fire_loads(ci, b):
        rbase = jnp.minimum(sid * ROWS_PER_TEC + ci * CROWS, E_ROWS - CROWS)
        col, row, val = bufs[b]
        pltpu.async_copy(cols2.at[pl.ds(rbase, CROWS)], col, sem)
        pltpu.async_copy(rows2.at[pl.ds(rbase, CROWS)], row, sem)
        pltpu.async_copy(vals2.at[pl.ds(rbase, CROWS)], val, sem)

    def drain_loads(b):
        col, row, val = bufs[b]
        pltpu.make_async_copy(cols2.at[pl.ds(0, CROWS)], col, sem).wait()
        pltpu.make_async_copy(rows2.at[pl.ds(0, CROWS)], row, sem).wait()
        pltpu.make_async_copy(vals2.at[pl.ds(0, CROWS)], val, sem).wait()

    def copy_rows(dst, src_):
        @plsc.parallel_loop(0, CROWS * 8, unroll=4)
        def cp(t):
            j = t // 8
            l = pl.multiple_of((t % 8) * LANES, LANES)
            dst[j, pl.ds(l, LANES)] = src_[j, pl.ds(l, LANES)]

    def edge_pass(src):
        def chunk_body(ci, b, first):
            colc, rowc, valc = bufs[b]
            coln = bufs[1 - b][0]
            S, Sp = srows[b], srows[1 - b]
            gd = []
            for j in range(CROWS):
                if not first:
                    pltpu.make_async_copy(rowsv0.at[pl.ds(j * IW, IW)],
                                          acc.at[Sp.at[j]], sems).wait()
                gd.append(pltpu.async_copy(src.at[colc.at[j]],
                                           rowsv0.at[pl.ds(j * IW, IW)], semg))
            drain_loads(1 - b)
            add_offset(coln, coln, CROWS, off)
            copy_rows(S, rowc)
            for j in range(CROWS):
                gd[j].wait()
                @plsc.parallel_loop(0, IW // LANES, unroll=4)
                def scale(g, j=j):
                    l = pl.multiple_of(g * LANES, LANES)
                    vv = valc[j, pl.ds(l, LANES)]
                    base_e = j * IW + g * LANES
                    for lane in range(LANES):
                        e = base_e + lane
                        rowsv0[e, :] = rowsv0[e, :] * vv[lane]
                pltpu.async_copy(rowsv0.at[pl.ds(j * IW, IW)],
                                 acc.at[S.at[j]], sems, add=True)
            fire_loads(ci + 2, b)

        fire_loads(0, 0)
        drain_loads(0)
        add_offset(colv0, colv0, CROWS, off)
        fire_loads(1, 1)
        chunk_body(0, 0, first=True)

        def chunk_pair(p, _):
            chunk_body(2 * p + 1, 1, first=False)
            chunk_body(2 * p + 2, 0, first=False)
            return 0
        lax.fori_loop(0, (NCHUNKS - 2) // 2, chunk_pair, 0)
        chunk_body(NCHUNKS - 1, 1, first=False)
        for j in range(CROWS):
            pltpu.make_async_copy(rowsv0.at[pl.ds(j * IW, IW)],
                                  acc.at[srow1.at[j]], sems).wait()
        drain_loads(1)

    def dump_stripe(dst):
        base = sid * STRIPE
        pltpu.sync_copy(acc.at[pl.ds(base, STRIPE)],
                        dst.at[pl.ds(off + base, STRIPE)])

    def batch_out():
        pltpu.sync_copy(u2.at[pl.ds(sid * BROWS, BROWS)], idxv)
        add_offset(iadj, idxv, BROWS, off)
        for j in range(BROWS):
            _final_block(j, ubuf)
        pltpu.sync_copy(i2.at[pl.ds(sid * BROWS, BROWS)], idxv)
        add_offset(iadj, idxv, BROWS, off)
        for j in range(BROWS):
            _final_block(j, ibuf)

    def _final_block(j, obuf):
        def accum():
            @plsc.parallel_loop(0, IW, unroll=4)
            def _(e):
                fbuf[e, :] = fbuf[e, :] + tmp[e, :]
        d0 = pltpu.async_copy(emb_s.at[iadj.at[j]], fbuf, semg)
        d1 = pltpu.async_copy(e1s.at[iadj.at[j]], tmp, sems)
        d0.wait()
        d1.wait()
        accum()
        pltpu.sync_copy(acc.at[idxv.at[j]], tmp)
        accum()
        obase = cid * BATCH + sid * BPT + j * IW
        pltpu.sync_copy(fbuf, obuf.at[pl.ds(obase, IW)])

    fill_zero_rowsv()
    zero_stripe()
    plsc.subcore_barrier()
    edge_pass(emb_s)
    plsc.subcore_barrier()
    dump_stripe(e1s)
    fill_zero_rowsv()
    zero_stripe()
    plsc.subcore_barrier()
    edge_pass(e1s)
    plsc.subcore_barrier()
    batch_out()


def kernel(user_ids, item_ids, node_emb, adj_row, adj_col, adj_vals):
    npad = N_PAD - N_TOTAL
    emb_s = jnp.concatenate(
        [jnp.pad(node_emb[:, :HALF], ((0, npad), (0, 0))),
         jnp.pad(node_emb[:, HALF:], ((0, npad), (0, 0)))], axis=0)
    pad = E_PAD - N_EDGES
    rows2 = jnp.pad(adj_row.astype(jnp.int32), (0, pad)).reshape(E_ROWS, IW)
    cols2 = jnp.pad(adj_col.astype(jnp.int32), (0, pad)).reshape(E_ROWS, IW)
    vals2 = jnp.pad(adj_vals, (0, pad)).reshape(E_ROWS, IW)
    u2 = user_ids.astype(jnp.int32).reshape(BATCH // IW, IW)
    i2 = (item_ids.astype(jnp.int32) + NUM_USERS).reshape(BATCH // IW, IW)
    ubuf, ibuf, _e1 = _sc_propagate(emb_s, rows2, cols2, vals2, u2, i2)
    part = pl.pallas_call(
        _dot_body,
        out_shape=jax.ShapeDtypeStruct((NC * BATCH,), jnp.float32),
    )(ubuf, ibuf)
    return part[:BATCH] + part[BATCH:]


def _dot_body(u_ref, i_ref, o_ref):
    o_ref[...] = jnp.sum(u_ref[...] * i_ref[...], axis=1) * (1.0 / 9.0)
